# baseline (speedup 1.0000x reference)
import functools

import jax
import jax.numpy as jnp
from jax import lax
from jax.experimental import pallas as pl
from jax.experimental.pallas import tpu as pltpu
from jax.experimental.pallas import tpu_sc as plsc

N_NODES = 10000
D_FEAT = 128
N_EDGES = 320000
N_LABEL = 100000

NC = 2
NS = 16
NW = NC * NS
CHUNK = 128
CPW = 80
E_PAD = NW * CPW * CHUNK
KB = 2
N_ACC = 10080
WBLK = 80
N_WBLK = N_ACC // WBLK
DUMMY = N_ACC - 1

L_PAD = 102400
DEC_CHUNKS = L_PAD // CHUNK
DEC_PER_W = DEC_CHUNKS // NW

_mesh = plsc.VectorSubcoreMesh(core_axis_name="c", subcore_axis_name="s")


def _wb_loop(sid, src_ref, dst_ref):
    n_blk = N_WBLK // NS + jnp.where(sid < N_WBLK % NS, 1, 0)

    def wb_body(t, _):
        off = pl.multiple_of((sid + t * NS) * WBLK, 16)
        pltpu.sync_copy(src_ref.at[pl.ds(off, WBLK)], dst_ref.at[pl.ds(off, WBLK)])
        return 0

    lax.fori_loop(0, n_blk, wb_body, 0)


HALF = CPW // 2


def _agg_sum_body(x_hbm, src_hbm, dst_hbm, z2d_hbm, agg_out,
                  idx_s, idx_d, rows, acc, sem_g, sem_s):
    cid = lax.axis_index("c")
    sid = lax.axis_index("s")
    wid = cid * NS + sid

    @pl.when(sid == 0)
    def _():
        pltpu.sync_copy(z2d_hbm, acc)

    plsc.subcore_barrier()

    def half_body(h, _):
        row0 = pl.multiple_of((wid * 2 + h) * HALF, 8)
        pltpu.sync_copy(src_hbm.at[pl.ds(row0 * CHUNK, HALF * CHUNK)], idx_s)
        pltpu.sync_copy(dst_hbm.at[pl.ds(row0, HALF)], idx_d)

        def block_body(tb, _):
            gathers = []
            for j in range(KB):
                c = tb * KB + j
                gathers.append(pltpu.async_copy(
                    x_hbm.at[idx_s.at[pl.ds(c * CHUNK, CHUNK)]],
                    rows.at[j], sem_g))
            for g in gathers:
                g.wait()
            scatters = [pltpu.async_copy(rows.at[j],
                                         acc.at[idx_d.at[tb * KB + j]],
                                         sem_s, add=True)
                        for j in range(KB)]
            for s in scatters:
                s.wait()
            return 0

        lax.fori_loop(0, HALF // KB, block_body, 0)
        return 0

    lax.fori_loop(0, 2, half_body, 0)
    plsc.subcore_barrier()
    _wb_loop(sid, acc, agg_out.at[cid])


_agg_sum = pl.kernel(
    _agg_sum_body,
    out_type=jax.ShapeDtypeStruct((NC, N_ACC, D_FEAT), jnp.float32),
    mesh=_mesh,
    scratch_types=[
        pltpu.VMEM((HALF * CHUNK,), jnp.int32),
        pltpu.VMEM((HALF, CHUNK), jnp.int32),
        pltpu.VMEM((KB, CHUNK, D_FEAT), jnp.float32),
        pltpu.VMEM_SHARED((N_ACC, D_FEAT), jnp.float32),
        pltpu.SemaphoreType.DMA,
        pltpu.SemaphoreType.DMA,
    ],
)


def _deg_sum_body(dst_hbm, z2d_hbm, ones_hbm, deg_out,
                  idx_d, ones_rows, acc, sem_s):
    cid = lax.axis_index("c")
    sid = lax.axis_index("s")
    wid = cid * NS + sid
    pltpu.sync_copy(ones_hbm, ones_rows)
    row0 = pl.multiple_of(wid * CPW, 8)
    pltpu.sync_copy(dst_hbm.at[pl.ds(row0, CPW)], idx_d)

    @pl.when(sid == 0)
    def _():
        pltpu.sync_copy(z2d_hbm, acc)

    plsc.subcore_barrier()

    def block_body(tb, _):
        scatters = [pltpu.async_copy(ones_rows, acc.at[idx_d.at[tb * 8 + j]],
                                     sem_s, add=True)
                    for j in range(8)]
        for s in scatters:
            s.wait()
        return 0

    lax.fori_loop(0, CPW // 8, block_body, 0)
    plsc.subcore_barrier()
    _wb_loop(sid, acc, deg_out.at[cid])


_deg_sum = pl.kernel(
    _deg_sum_body,
    out_type=jax.ShapeDtypeStruct((NC, N_ACC, D_FEAT), jnp.float32),
    mesh=_mesh,
    scratch_types=[
        pltpu.VMEM((CPW, CHUNK), jnp.int32),
        pltpu.VMEM((CHUNK, D_FEAT), jnp.float32),
        pltpu.VMEM_SHARED((N_ACC, D_FEAT), jnp.float32),
        pltpu.SemaphoreType.DMA,
    ],
)


def _decode_body(z_hbm, s_hbm, d_hbm, gs_out, gd_out,
                 idx_a, idx_b, rows_a, rows_b,
                 sem_g0, sem_g1, sem_w0, sem_w1):
    cid = lax.axis_index("c")
    sid = lax.axis_index("s")
    wid = cid * NS + sid
    nb = DEC_PER_W * CHUNK
    pltpu.sync_copy(s_hbm.at[pl.ds(wid * nb, nb)], idx_a)
    pltpu.sync_copy(d_hbm.at[pl.ds(wid * nb, nb)], idx_b)

    sem_g = [sem_g0, sem_g1]
    sem_w = [sem_w0, sem_w1]
    pending_wb = [None, None]

    for t in range(DEC_PER_W):
        slot = t % 2
        if pending_wb[slot] is not None:
            for w in pending_wb[slot]:
                w.wait()
        ga = pltpu.async_copy(z_hbm.at[idx_a.at[pl.ds(t * CHUNK, CHUNK)]],
                              rows_a.at[slot], sem_g[slot])
        gb = pltpu.async_copy(z_hbm.at[idx_b.at[pl.ds(t * CHUNK, CHUNK)]],
                              rows_b.at[slot], sem_g[slot])
        ga.wait()
        gb.wait()
        base = (wid * DEC_PER_W + t) * CHUNK
        wa = pltpu.async_copy(rows_a.at[slot], gs_out.at[pl.ds(base, CHUNK)],
                              sem_w[slot])
        wb = pltpu.async_copy(rows_b.at[slot], gd_out.at[pl.ds(base, CHUNK)],
                              sem_w[slot])
        pending_wb[slot] = (wa, wb)

    for p in pending_wb:
        if p is not None:
            for w in p:
                w.wait()


_decode = pl.kernel(
    _decode_body,
    out_type=(jax.ShapeDtypeStruct((L_PAD, D_FEAT), jnp.float32),
              jax.ShapeDtypeStruct((L_PAD, D_FEAT), jnp.float32)),
    mesh=_mesh,
    scratch_types=[
        pltpu.VMEM((DEC_PER_W * CHUNK,), jnp.int32),
        pltpu.VMEM((DEC_PER_W * CHUNK,), jnp.int32),
        pltpu.VMEM((2, CHUNK, D_FEAT), jnp.float32),
        pltpu.VMEM((2, CHUNK, D_FEAT), jnp.float32),
        pltpu.SemaphoreType.DMA,
        pltpu.SemaphoreType.DMA,
        pltpu.SemaphoreType.DMA,
        pltpu.SemaphoreType.DMA,
    ],
)


def _rowsum_body(gs_ref, gd_ref, out_ref):
    out_ref[...] = jnp.sum(gs_ref[...] * gd_ref[...], axis=1)


def _rowsum_tc(gs, gd):
    rows_blk = 10240
    return pl.pallas_call(
        _rowsum_body,
        grid=(L_PAD // rows_blk,),
        in_specs=[pl.BlockSpec((rows_blk, D_FEAT), lambda i: (i, 0)),
                  pl.BlockSpec((rows_blk, D_FEAT), lambda i: (i, 0))],
        out_specs=pl.BlockSpec((rows_blk,), lambda i: (i,)),
        out_shape=jax.ShapeDtypeStruct((L_PAD,), jnp.float32),
    )(gs, gd)


def _layer_tc_body(relu, agg_ref, deg_ref, w_ref, b_ref, out_ref):
    part = agg_ref[0] + agg_ref[1]
    deg = jnp.sum(deg_ref[0] + deg_ref[1], axis=1, keepdims=True)
    deg = jnp.maximum(deg, 1.0)
    aggn = part / deg
    y = jnp.dot(aggn, w_ref[...], preferred_element_type=jnp.float32) + b_ref[...]
    if relu:
        y = jnp.maximum(y, 0.0)
    out_ref[...] = y


def _layer_tc(agg_part, deg_part, w, b, relu):
    rows_blk = 2016
    grid = (N_ACC // rows_blk,)
    return pl.pallas_call(
        functools.partial(_layer_tc_body, relu),
        grid=grid,
        in_specs=[
            pl.BlockSpec((NC, rows_blk, D_FEAT), lambda i: (0, i, 0)),
            pl.BlockSpec((NC, rows_blk, D_FEAT), lambda i: (0, i, 0)),
            pl.BlockSpec((D_FEAT, D_FEAT), lambda i: (0, 0)),
            pl.BlockSpec((1, D_FEAT), lambda i: (0, 0)),
        ],
        out_specs=pl.BlockSpec((rows_blk, D_FEAT), lambda i: (i, 0)),
        out_shape=jax.ShapeDtypeStruct((N_ACC, D_FEAT), jnp.float32),
    )(agg_part, deg_part, w, b)


def kernel(x, edge_index, edge_label_index, W1, b1, W2, b2):
    src_pad = jnp.zeros((E_PAD,), jnp.int32).at[:N_EDGES].set(edge_index[0])
    dst_pad = jnp.full((E_PAD,), DUMMY, jnp.int32).at[:N_EDGES].set(edge_index[1])
    dst_r = dst_pad.reshape(NW * CPW, CHUNK)
    s_r = jnp.zeros((L_PAD,), jnp.int32).at[:N_LABEL].set(edge_label_index[0])
    d_r = jnp.zeros((L_PAD,), jnp.int32).at[:N_LABEL].set(edge_label_index[1])
    x_pad = jnp.zeros((N_ACC, D_FEAT), jnp.float32).at[:N_NODES].set(x)
    z2d = jnp.zeros((N_ACC, D_FEAT), jnp.float32)
    ones2d = jnp.zeros((CHUNK, D_FEAT), jnp.float32).at[:, 0].set(1.0)

    degp = _deg_sum(dst_r, z2d, ones2d)
    agg1p = _agg_sum(x_pad, src_pad, dst_r, z2d)
    h = _layer_tc(agg1p, degp, W1, b1.reshape(1, D_FEAT), relu=True)
    agg2p = _agg_sum(h, src_pad, dst_r, z2d)
    z = _layer_tc(agg2p, degp, W2, b2.reshape(1, D_FEAT), relu=False)
    gs, gd = _decode(z, s_r, d_r)
    scores_pad = _rowsum_tc(gs, gd)
    return scores_pad[:N_LABEL]


# speedup vs baseline: 2.5563x; 2.5563x over previous
import functools

import jax
import jax.numpy as jnp
from jax import lax
from jax.experimental import pallas as pl
from jax.experimental.pallas import tpu as pltpu
from jax.experimental.pallas import tpu_sc as plsc

N_NODES = 10000
D_FEAT = 128
N_EDGES = 320000
N_LABEL = 100000

NC = 2
NS = 16
NW = NC * NS
CHUNK = 128
CPW = 80
E_PAD = NW * CPW * CHUNK
KB = 2
N_ACC = 10080
WBLK = 80
N_WBLK = N_ACC // WBLK
DUMMY = N_ACC - 1

L_PAD = 102400
DEC_CHUNKS = L_PAD // CHUNK
DEC_PER_W = DEC_CHUNKS // NW

_mesh = plsc.VectorSubcoreMesh(core_axis_name="c", subcore_axis_name="s")


def _wb_loop(sid, src_ref, dst_ref):
    n_blk = N_WBLK // NS + jnp.where(sid < N_WBLK % NS, 1, 0)

    def wb_body(t, _):
        off = pl.multiple_of((sid + t * NS) * WBLK, 16)
        pltpu.sync_copy(src_ref.at[pl.ds(off, WBLK)], dst_ref.at[pl.ds(off, WBLK)])
        return 0

    lax.fori_loop(0, n_blk, wb_body, 0)


HALF = CPW // 2


def _agg_sum_body(x_hbm, src_hbm, dst_hbm, z2d_hbm, agg_out,
                  idx_s, idx_d, rows, acc, sem_g, sem_s):
    cid = lax.axis_index("c")
    sid = lax.axis_index("s")
    wid = cid * NS + sid

    @pl.when(sid == 0)
    def _():
        pltpu.sync_copy(z2d_hbm, acc)

    plsc.subcore_barrier()

    def half_body(h, _):
        row0 = pl.multiple_of((wid * 2 + h) * HALF, 8)
        pltpu.sync_copy(src_hbm.at[pl.ds(row0 * CHUNK, HALF * CHUNK)], idx_s)
        pltpu.sync_copy(dst_hbm.at[pl.ds(row0, HALF)], idx_d)

        def block_body(tb, _):
            gathers = []
            for j in range(KB):
                c = tb * KB + j
                gathers.append(pltpu.async_copy(
                    x_hbm.at[idx_s.at[pl.ds(c * CHUNK, CHUNK)]],
                    rows.at[j], sem_g))
            for g in gathers:
                g.wait()
            scatters = [pltpu.async_copy(rows.at[j],
                                         acc.at[idx_d.at[tb * KB + j]],
                                         sem_s, add=True)
                        for j in range(KB)]
            for s in scatters:
                s.wait()
            return 0

        lax.fori_loop(0, HALF // KB, block_body, 0)
        return 0

    lax.fori_loop(0, 2, half_body, 0)
    plsc.subcore_barrier()
    _wb_loop(sid, acc, agg_out.at[cid])


_agg_sum = pl.kernel(
    _agg_sum_body,
    out_type=jax.ShapeDtypeStruct((NC, N_ACC, D_FEAT), jnp.float32),
    mesh=_mesh,
    scratch_types=[
        pltpu.VMEM((HALF * CHUNK,), jnp.int32),
        pltpu.VMEM((HALF, CHUNK), jnp.int32),
        pltpu.VMEM((KB, CHUNK, D_FEAT), jnp.float32),
        pltpu.VMEM_SHARED((N_ACC, D_FEAT), jnp.float32),
        pltpu.SemaphoreType.DMA,
        pltpu.SemaphoreType.DMA,
    ],
)


def _deg_sum_body(dst_hbm, z2d_hbm, ones_hbm, deg_out,
                  idx_d, ones_rows, acc, sem_s):
    cid = lax.axis_index("c")
    sid = lax.axis_index("s")
    wid = cid * NS + sid
    pltpu.sync_copy(ones_hbm, ones_rows)
    row0 = pl.multiple_of(wid * CPW, 8)
    pltpu.sync_copy(dst_hbm.at[pl.ds(row0, CPW)], idx_d)

    @pl.when(sid == 0)
    def _():
        pltpu.sync_copy(z2d_hbm, acc)

    plsc.subcore_barrier()

    def block_body(tb, _):
        scatters = [pltpu.async_copy(ones_rows, acc.at[idx_d.at[tb * 8 + j]],
                                     sem_s, add=True)
                    for j in range(8)]
        for s in scatters:
            s.wait()
        return 0

    lax.fori_loop(0, CPW // 8, block_body, 0)
    plsc.subcore_barrier()
    _wb_loop(sid, acc, deg_out.at[cid])


_deg_sum = pl.kernel(
    _deg_sum_body,
    out_type=jax.ShapeDtypeStruct((NC, N_ACC, D_FEAT), jnp.float32),
    mesh=_mesh,
    scratch_types=[
        pltpu.VMEM((CPW, CHUNK), jnp.int32),
        pltpu.VMEM((CHUNK, D_FEAT), jnp.float32),
        pltpu.VMEM_SHARED((N_ACC, D_FEAT), jnp.float32),
        pltpu.SemaphoreType.DMA,
    ],
)


def _decode_body(z_hbm, s_hbm, d_hbm, gs_out, gd_out,
                 idx_a, idx_b, rows_a, rows_b,
                 sem_g0, sem_g1, sem_w0, sem_w1):
    cid = lax.axis_index("c")
    sid = lax.axis_index("s")
    wid = cid * NS + sid
    nb = DEC_PER_W * CHUNK
    pltpu.sync_copy(s_hbm.at[pl.ds(wid * nb, nb)], idx_a)
    pltpu.sync_copy(d_hbm.at[pl.ds(wid * nb, nb)], idx_b)

    sem_g = [sem_g0, sem_g1]
    sem_w = [sem_w0, sem_w1]
    pending_wb = [None, None]

    for t in range(DEC_PER_W):
        slot = t % 2
        if pending_wb[slot] is not None:
            for w in pending_wb[slot]:
                w.wait()
        ga = pltpu.async_copy(z_hbm.at[idx_a.at[pl.ds(t * CHUNK, CHUNK)]],
                              rows_a.at[slot], sem_g[slot])
        gb = pltpu.async_copy(z_hbm.at[idx_b.at[pl.ds(t * CHUNK, CHUNK)]],
                              rows_b.at[slot], sem_g[slot])
        ga.wait()
        gb.wait()
        base = (wid * DEC_PER_W + t) * CHUNK
        wa = pltpu.async_copy(rows_a.at[slot], gs_out.at[pl.ds(base, CHUNK)],
                              sem_w[slot])
        wb = pltpu.async_copy(rows_b.at[slot], gd_out.at[pl.ds(base, CHUNK)],
                              sem_w[slot])
        pending_wb[slot] = (wa, wb)

    for p in pending_wb:
        if p is not None:
            for w in p:
                w.wait()


_decode = pl.kernel(
    _decode_body,
    out_type=(jax.ShapeDtypeStruct((L_PAD, D_FEAT), jnp.float32),
              jax.ShapeDtypeStruct((L_PAD, D_FEAT), jnp.float32)),
    mesh=_mesh,
    scratch_types=[
        pltpu.VMEM((DEC_PER_W * CHUNK,), jnp.int32),
        pltpu.VMEM((DEC_PER_W * CHUNK,), jnp.int32),
        pltpu.VMEM((2, CHUNK, D_FEAT), jnp.float32),
        pltpu.VMEM((2, CHUNK, D_FEAT), jnp.float32),
        pltpu.SemaphoreType.DMA,
        pltpu.SemaphoreType.DMA,
        pltpu.SemaphoreType.DMA,
        pltpu.SemaphoreType.DMA,
    ],
)


def _rowsum_body(gs_ref, gd_ref, out_ref):
    out_ref[...] = jnp.sum(gs_ref[...] * gd_ref[...], axis=1)


def _rowsum_tc(gs, gd):
    rows_blk = 10240
    return pl.pallas_call(
        _rowsum_body,
        grid=(L_PAD // rows_blk,),
        in_specs=[pl.BlockSpec((rows_blk, D_FEAT), lambda i: (i, 0)),
                  pl.BlockSpec((rows_blk, D_FEAT), lambda i: (i, 0))],
        out_specs=pl.BlockSpec((rows_blk,), lambda i: (i,)),
        out_shape=jax.ShapeDtypeStruct((L_PAD,), jnp.float32),
    )(gs, gd)


def _layer_tc_body(relu, agg_ref, deg_ref, w_ref, b_ref, out_ref):
    part = agg_ref[0] + agg_ref[1]
    deg = jnp.sum(deg_ref[0] + deg_ref[1], axis=1, keepdims=True)
    deg = jnp.maximum(deg, 1.0)
    aggn = part / deg
    y = jnp.dot(aggn, w_ref[...], preferred_element_type=jnp.float32) + b_ref[...]
    if relu:
        y = jnp.maximum(y, 0.0)
    out_ref[...] = y


def _layer_tc(agg_part, deg_part, w, b, relu):
    rows_blk = 2016
    grid = (N_ACC // rows_blk,)
    return pl.pallas_call(
        functools.partial(_layer_tc_body, relu),
        grid=grid,
        in_specs=[
            pl.BlockSpec((NC, rows_blk, D_FEAT), lambda i: (0, i, 0)),
            pl.BlockSpec((NC, rows_blk, D_FEAT), lambda i: (0, i, 0)),
            pl.BlockSpec((D_FEAT, D_FEAT), lambda i: (0, 0)),
            pl.BlockSpec((1, D_FEAT), lambda i: (0, 0)),
        ],
        out_specs=pl.BlockSpec((rows_blk, D_FEAT), lambda i: (i, 0)),
        out_shape=jax.ShapeDtypeStruct((N_ACC, D_FEAT), jnp.float32),
    )(agg_part, deg_part, w, b)


def kernel(x, edge_index, edge_label_index, W1, b1, W2, b2):
    epad = jnp.arange(E_PAD - N_EDGES, dtype=jnp.int32)
    lpad = jnp.arange(L_PAD - N_LABEL, dtype=jnp.int32)
    src_pad = jnp.concatenate([edge_index[0].astype(jnp.int32), epad % N_NODES])
    dst_pad = jnp.concatenate([edge_index[1].astype(jnp.int32),
                               N_NODES + epad % (N_ACC - N_NODES)])
    dst_r = dst_pad.reshape(NW * CPW, CHUNK)
    s_r = jnp.concatenate([edge_label_index[0].astype(jnp.int32), lpad % N_NODES])
    d_r = jnp.concatenate([edge_label_index[1].astype(jnp.int32), lpad % N_NODES])
    x_pad = jnp.zeros((N_ACC, D_FEAT), jnp.float32).at[:N_NODES].set(x)
    z2d = jnp.zeros((N_ACC, D_FEAT), jnp.float32)
    ones2d = jnp.zeros((CHUNK, D_FEAT), jnp.float32).at[:, 0].set(1.0)

    degp = _deg_sum(dst_r, z2d, ones2d)
    agg1p = _agg_sum(x_pad, src_pad, dst_r, z2d)
    h = _layer_tc(agg1p, degp, W1, b1.reshape(1, D_FEAT), relu=True)
    agg2p = _agg_sum(h, src_pad, dst_r, z2d)
    z = _layer_tc(agg2p, degp, W2, b2.reshape(1, D_FEAT), relu=False)
    gs, gd = _decode(z, s_r, d_r)
    scores_pad = _rowsum_tc(gs, gd)
    return scores_pad[:N_LABEL]


# speedup vs baseline: 2.9396x; 1.1499x over previous
import functools

import jax
import jax.numpy as jnp
from jax import lax
from jax.experimental import pallas as pl
from jax.experimental.pallas import tpu as pltpu
from jax.experimental.pallas import tpu_sc as plsc

N_NODES = 10000
D_FEAT = 128
N_EDGES = 320000
N_LABEL = 100000

NC = 2
NS = 16
NW = NC * NS
CHUNK = 128
CPW = 80
E_PAD = NW * CPW * CHUNK
KB = 2
N_ACC = 10080
WBLK = 80
N_WBLK = N_ACC // WBLK
DUMMY = N_ACC - 1

L_PAD = 102400
DEC_CHUNKS = L_PAD // CHUNK
DEC_PER_W = DEC_CHUNKS // NW

_mesh = plsc.VectorSubcoreMesh(core_axis_name="c", subcore_axis_name="s")


def _wb_loop(sid, src_ref, dst_ref):
    n_blk = N_WBLK // NS + jnp.where(sid < N_WBLK % NS, 1, 0)

    def wb_body(t, _):
        off = pl.multiple_of((sid + t * NS) * WBLK, 16)
        pltpu.sync_copy(src_ref.at[pl.ds(off, WBLK)], dst_ref.at[pl.ds(off, WBLK)])
        return 0

    lax.fori_loop(0, n_blk, wb_body, 0)


HALF = CPW // 2


def _agg_sum_body(x_hbm, src_hbm, dst_hbm, z2d_hbm, agg_out,
                  idx_s, idx_d, rows, acc, sem_g, sem_s):
    cid = lax.axis_index("c")
    sid = lax.axis_index("s")
    wid = cid * NS + sid

    @pl.when(sid == 0)
    def _():
        pltpu.sync_copy(z2d_hbm, acc)

    plsc.subcore_barrier()

    def gather(c, j):
        return pltpu.async_copy(x_hbm.at[idx_s.at[pl.ds(c * CHUNK, CHUNK)]],
                                rows.at[j], sem_g)

    def scatter(c, j):
        return pltpu.async_copy(rows.at[j], acc.at[idx_d.at[c]], sem_s,
                                add=True)

    def drain_scatter(j):
        pltpu.make_async_copy(rows.at[j], acc.at[idx_d.at[0]], sem_s).wait()

    def half_body(h, _):
        row0 = pl.multiple_of((wid * 2 + h) * HALF, 8)
        pltpu.sync_copy(src_hbm.at[pl.ds(row0 * CHUNK, HALF * CHUNK)], idx_s)
        pltpu.sync_copy(dst_hbm.at[pl.ds(row0, HALF)], idx_d)

        g0 = gather(0, 0)
        g1 = gather(1, 1)
        g0.wait()
        scatter(0, 0)
        g1.wait()
        scatter(1, 1)

        def block_body(tb, _):
            c0 = tb * KB
            drain_scatter(0)
            ga = gather(c0, 0)
            drain_scatter(1)
            gb = gather(c0 + 1, 1)
            ga.wait()
            scatter(c0, 0)
            gb.wait()
            scatter(c0 + 1, 1)
            return 0

        lax.fori_loop(1, HALF // KB, block_body, 0)
        drain_scatter(0)
        drain_scatter(1)
        return 0

    lax.fori_loop(0, 2, half_body, 0)
    plsc.subcore_barrier()
    _wb_loop(sid, acc, agg_out.at[cid])


_agg_sum = pl.kernel(
    _agg_sum_body,
    out_type=jax.ShapeDtypeStruct((NC, N_ACC, D_FEAT), jnp.float32),
    mesh=_mesh,
    scratch_types=[
        pltpu.VMEM((HALF * CHUNK,), jnp.int32),
        pltpu.VMEM((HALF, CHUNK), jnp.int32),
        pltpu.VMEM((KB, CHUNK, D_FEAT), jnp.float32),
        pltpu.VMEM_SHARED((N_ACC, D_FEAT), jnp.float32),
        pltpu.SemaphoreType.DMA,
        pltpu.SemaphoreType.DMA,
    ],
)


def _deg_sum_body(dst_hbm, z2d_hbm, ones_hbm, deg_out,
                  idx_d, ones_rows, acc, sem_s):
    cid = lax.axis_index("c")
    sid = lax.axis_index("s")
    wid = cid * NS + sid
    pltpu.sync_copy(ones_hbm, ones_rows)
    row0 = pl.multiple_of(wid * CPW, 8)
    pltpu.sync_copy(dst_hbm.at[pl.ds(row0, CPW)], idx_d)

    @pl.when(sid == 0)
    def _():
        pltpu.sync_copy(z2d_hbm, acc)

    plsc.subcore_barrier()

    def block_body(tb, _):
        scatters = [pltpu.async_copy(ones_rows, acc.at[idx_d.at[tb * 8 + j]],
                                     sem_s, add=True)
                    for j in range(8)]
        for s in scatters:
            s.wait()
        return 0

    lax.fori_loop(0, CPW // 8, block_body, 0)
    plsc.subcore_barrier()
    _wb_loop(sid, acc, deg_out.at[cid])


_deg_sum = pl.kernel(
    _deg_sum_body,
    out_type=jax.ShapeDtypeStruct((NC, N_ACC, D_FEAT), jnp.float32),
    mesh=_mesh,
    scratch_types=[
        pltpu.VMEM((CPW, CHUNK), jnp.int32),
        pltpu.VMEM((CHUNK, D_FEAT), jnp.float32),
        pltpu.VMEM_SHARED((N_ACC, D_FEAT), jnp.float32),
        pltpu.SemaphoreType.DMA,
    ],
)


def _decode_body(z_hbm, s_hbm, d_hbm, gs_out, gd_out,
                 idx_a, idx_b, rows_a, rows_b,
                 sem_g0, sem_g1, sem_w0, sem_w1):
    cid = lax.axis_index("c")
    sid = lax.axis_index("s")
    wid = cid * NS + sid
    nb = DEC_PER_W * CHUNK
    pltpu.sync_copy(s_hbm.at[pl.ds(wid * nb, nb)], idx_a)
    pltpu.sync_copy(d_hbm.at[pl.ds(wid * nb, nb)], idx_b)

    sem_g = [sem_g0, sem_g1]
    sem_w = [sem_w0, sem_w1]
    pending_wb = [None, None]

    for t in range(DEC_PER_W):
        slot = t % 2
        if pending_wb[slot] is not None:
            for w in pending_wb[slot]:
                w.wait()
        ga = pltpu.async_copy(z_hbm.at[idx_a.at[pl.ds(t * CHUNK, CHUNK)]],
                              rows_a.at[slot], sem_g[slot])
        gb = pltpu.async_copy(z_hbm.at[idx_b.at[pl.ds(t * CHUNK, CHUNK)]],
                              rows_b.at[slot], sem_g[slot])
        ga.wait()
        gb.wait()
        base = (wid * DEC_PER_W + t) * CHUNK
        wa = pltpu.async_copy(rows_a.at[slot], gs_out.at[pl.ds(base, CHUNK)],
                              sem_w[slot])
        wb = pltpu.async_copy(rows_b.at[slot], gd_out.at[pl.ds(base, CHUNK)],
                              sem_w[slot])
        pending_wb[slot] = (wa, wb)

    for p in pending_wb:
        if p is not None:
            for w in p:
                w.wait()


_decode = pl.kernel(
    _decode_body,
    out_type=(jax.ShapeDtypeStruct((L_PAD, D_FEAT), jnp.float32),
              jax.ShapeDtypeStruct((L_PAD, D_FEAT), jnp.float32)),
    mesh=_mesh,
    scratch_types=[
        pltpu.VMEM((DEC_PER_W * CHUNK,), jnp.int32),
        pltpu.VMEM((DEC_PER_W * CHUNK,), jnp.int32),
        pltpu.VMEM((2, CHUNK, D_FEAT), jnp.float32),
        pltpu.VMEM((2, CHUNK, D_FEAT), jnp.float32),
        pltpu.SemaphoreType.DMA,
        pltpu.SemaphoreType.DMA,
        pltpu.SemaphoreType.DMA,
        pltpu.SemaphoreType.DMA,
    ],
)


def _rowsum_body(gs_ref, gd_ref, out_ref):
    out_ref[...] = jnp.sum(gs_ref[...] * gd_ref[...], axis=1)


def _rowsum_tc(gs, gd):
    rows_blk = 10240
    return pl.pallas_call(
        _rowsum_body,
        grid=(L_PAD // rows_blk,),
        in_specs=[pl.BlockSpec((rows_blk, D_FEAT), lambda i: (i, 0)),
                  pl.BlockSpec((rows_blk, D_FEAT), lambda i: (i, 0))],
        out_specs=pl.BlockSpec((rows_blk,), lambda i: (i,)),
        out_shape=jax.ShapeDtypeStruct((L_PAD,), jnp.float32),
    )(gs, gd)


def _layer_tc_body(relu, agg_ref, deg_ref, w_ref, b_ref, out_ref):
    part = agg_ref[0] + agg_ref[1]
    deg = jnp.sum(deg_ref[0] + deg_ref[1], axis=1, keepdims=True)
    deg = jnp.maximum(deg, 1.0)
    aggn = part / deg
    y = jnp.dot(aggn, w_ref[...], preferred_element_type=jnp.float32) + b_ref[...]
    if relu:
        y = jnp.maximum(y, 0.0)
    out_ref[...] = y


def _layer_tc(agg_part, deg_part, w, b, relu):
    rows_blk = 2016
    grid = (N_ACC // rows_blk,)
    return pl.pallas_call(
        functools.partial(_layer_tc_body, relu),
        grid=grid,
        in_specs=[
            pl.BlockSpec((NC, rows_blk, D_FEAT), lambda i: (0, i, 0)),
            pl.BlockSpec((NC, rows_blk, D_FEAT), lambda i: (0, i, 0)),
            pl.BlockSpec((D_FEAT, D_FEAT), lambda i: (0, 0)),
            pl.BlockSpec((1, D_FEAT), lambda i: (0, 0)),
        ],
        out_specs=pl.BlockSpec((rows_blk, D_FEAT), lambda i: (i, 0)),
        out_shape=jax.ShapeDtypeStruct((N_ACC, D_FEAT), jnp.float32),
    )(agg_part, deg_part, w, b)


def kernel(x, edge_index, edge_label_index, W1, b1, W2, b2):
    epad = jnp.arange(E_PAD - N_EDGES, dtype=jnp.int32)
    lpad = jnp.arange(L_PAD - N_LABEL, dtype=jnp.int32)
    src_pad = jnp.concatenate([edge_index[0].astype(jnp.int32), epad % N_NODES])
    dst_pad = jnp.concatenate([edge_index[1].astype(jnp.int32),
                               N_NODES + epad % (N_ACC - N_NODES)])
    dst_r = dst_pad.reshape(NW * CPW, CHUNK)
    s_r = jnp.concatenate([edge_label_index[0].astype(jnp.int32), lpad % N_NODES])
    d_r = jnp.concatenate([edge_label_index[1].astype(jnp.int32), lpad % N_NODES])
    x_pad = jnp.zeros((N_ACC, D_FEAT), jnp.float32).at[:N_NODES].set(x)
    z2d = jnp.zeros((N_ACC, D_FEAT), jnp.float32)
    ones2d = jnp.zeros((CHUNK, D_FEAT), jnp.float32).at[:, 0].set(1.0)

    degp = _deg_sum(dst_r, z2d, ones2d)
    agg1p = _agg_sum(x_pad, src_pad, dst_r, z2d)
    h = _layer_tc(agg1p, degp, W1, b1.reshape(1, D_FEAT), relu=True)
    agg2p = _agg_sum(h, src_pad, dst_r, z2d)
    z = _layer_tc(agg2p, degp, W2, b2.reshape(1, D_FEAT), relu=False)
    gs, gd = _decode(z, s_r, d_r)
    scores_pad = _rowsum_tc(gs, gd)
    return scores_pad[:N_LABEL]


# speedup vs baseline: 2.9404x; 1.0003x over previous
import functools

import jax
import jax.numpy as jnp
from jax import lax
from jax.experimental import pallas as pl
from jax.experimental.pallas import tpu as pltpu
from jax.experimental.pallas import tpu_sc as plsc

N_NODES = 10000
D_FEAT = 128
N_EDGES = 320000
N_LABEL = 100000

NC = 2
NS = 16
NW = NC * NS
CHUNK = 128
CPW = 80
E_PAD = NW * CPW * CHUNK
KB = 2
N_ACC = 10080
WBLK = 80
N_WBLK = N_ACC // WBLK
DUMMY = N_ACC - 1

L_PAD = 102400
DEC_CHUNKS = L_PAD // CHUNK
DEC_PER_W = DEC_CHUNKS // NW

_mesh = plsc.VectorSubcoreMesh(core_axis_name="c", subcore_axis_name="s")


def _wb_loop(sid, src_ref, dst_ref):
    n_blk = N_WBLK // NS + jnp.where(sid < N_WBLK % NS, 1, 0)

    def wb_body(t, _):
        off = pl.multiple_of((sid + t * NS) * WBLK, 16)
        pltpu.sync_copy(src_ref.at[pl.ds(off, WBLK)], dst_ref.at[pl.ds(off, WBLK)])
        return 0

    lax.fori_loop(0, n_blk, wb_body, 0)


HALF = CPW // 2


def _agg_sum_body(x_hbm, src_hbm, dst_hbm, z2d_hbm, agg_out,
                  idx_s, idx_d, rows, acc, sem_g, sem_s):
    cid = lax.axis_index("c")
    sid = lax.axis_index("s")
    wid = cid * NS + sid

    @pl.when(sid == 0)
    def _():
        pltpu.sync_copy(z2d_hbm, acc)

    plsc.subcore_barrier()

    def gather(c, j):
        return pltpu.async_copy(x_hbm.at[idx_s.at[pl.ds(c * CHUNK, CHUNK)]],
                                rows.at[j], sem_g)

    def scatter(c, j):
        return pltpu.async_copy(rows.at[j], acc.at[idx_d.at[c]], sem_s,
                                add=True)

    def drain_scatter(j):
        pltpu.make_async_copy(rows.at[j], acc.at[idx_d.at[0]], sem_s).wait()

    def half_body(h, _):
        row0 = pl.multiple_of((wid * 2 + h) * HALF, 8)
        pltpu.sync_copy(src_hbm.at[pl.ds(row0 * CHUNK, HALF * CHUNK)], idx_s)
        pltpu.sync_copy(dst_hbm.at[pl.ds(row0, HALF)], idx_d)

        g0 = gather(0, 0)
        g1 = gather(1, 1)
        g0.wait()
        scatter(0, 0)
        g1.wait()
        scatter(1, 1)

        def block_body(tb, _):
            c0 = tb * KB
            drain_scatter(0)
            ga = gather(c0, 0)
            drain_scatter(1)
            gb = gather(c0 + 1, 1)
            ga.wait()
            scatter(c0, 0)
            gb.wait()
            scatter(c0 + 1, 1)
            return 0

        lax.fori_loop(1, HALF // KB, block_body, 0)
        drain_scatter(0)
        drain_scatter(1)
        return 0

    lax.fori_loop(0, 2, half_body, 0)
    plsc.subcore_barrier()
    _wb_loop(sid, acc, agg_out.at[cid])


_agg_sum = pl.kernel(
    _agg_sum_body,
    out_type=jax.ShapeDtypeStruct((NC, N_ACC, D_FEAT), jnp.float32),
    mesh=_mesh,
    scratch_types=[
        pltpu.VMEM((HALF * CHUNK,), jnp.int32),
        pltpu.VMEM((HALF, CHUNK), jnp.int32),
        pltpu.VMEM((KB, CHUNK, D_FEAT), jnp.float32),
        pltpu.VMEM_SHARED((N_ACC, D_FEAT), jnp.float32),
        pltpu.SemaphoreType.DMA,
        pltpu.SemaphoreType.DMA,
    ],
)


def _deg_sum_body(dst_hbm, z2d_hbm, ones_hbm, deg_out,
                  idx_d, ones_rows, acc, sem_s):
    cid = lax.axis_index("c")
    sid = lax.axis_index("s")
    wid = cid * NS + sid
    pltpu.sync_copy(ones_hbm, ones_rows)
    row0 = pl.multiple_of(wid * CPW, 8)
    pltpu.sync_copy(dst_hbm.at[pl.ds(row0, CPW)], idx_d)

    @pl.when(sid == 0)
    def _():
        pltpu.sync_copy(z2d_hbm, acc)

    plsc.subcore_barrier()

    for j in range(8):
        pltpu.async_copy(ones_rows, acc.at[idx_d.at[j]], sem_s, add=True)

    def block_body(tb, _):
        for j in range(8):
            pltpu.make_async_copy(ones_rows, acc.at[idx_d.at[0]], sem_s).wait()
            pltpu.async_copy(ones_rows, acc.at[idx_d.at[tb * 8 + j]],
                             sem_s, add=True)
        return 0

    lax.fori_loop(1, CPW // 8, block_body, 0)
    for j in range(8):
        pltpu.make_async_copy(ones_rows, acc.at[idx_d.at[0]], sem_s).wait()
    plsc.subcore_barrier()
    _wb_loop(sid, acc, deg_out.at[cid])


_deg_sum = pl.kernel(
    _deg_sum_body,
    out_type=jax.ShapeDtypeStruct((NC, N_ACC, D_FEAT), jnp.float32),
    mesh=_mesh,
    scratch_types=[
        pltpu.VMEM((CPW, CHUNK), jnp.int32),
        pltpu.VMEM((CHUNK, D_FEAT), jnp.float32),
        pltpu.VMEM_SHARED((N_ACC, D_FEAT), jnp.float32),
        pltpu.SemaphoreType.DMA,
    ],
)


def _decode_body(z_hbm, s_hbm, d_hbm, gs_out, gd_out,
                 idx_a, idx_b, rows_a, rows_b,
                 sem_g0, sem_g1, sem_w0, sem_w1):
    cid = lax.axis_index("c")
    sid = lax.axis_index("s")
    wid = cid * NS + sid
    nb = DEC_PER_W * CHUNK
    pltpu.sync_copy(s_hbm.at[pl.ds(wid * nb, nb)], idx_a)
    pltpu.sync_copy(d_hbm.at[pl.ds(wid * nb, nb)], idx_b)

    sem_g = [sem_g0, sem_g1]
    sem_w = [sem_w0, sem_w1]
    pending_wb = [None, None]

    for t in range(DEC_PER_W):
        slot = t % 2
        if pending_wb[slot] is not None:
            for w in pending_wb[slot]:
                w.wait()
        ga = pltpu.async_copy(z_hbm.at[idx_a.at[pl.ds(t * CHUNK, CHUNK)]],
                              rows_a.at[slot], sem_g[slot])
        gb = pltpu.async_copy(z_hbm.at[idx_b.at[pl.ds(t * CHUNK, CHUNK)]],
                              rows_b.at[slot], sem_g[slot])
        ga.wait()
        gb.wait()
        base = (wid * DEC_PER_W + t) * CHUNK
        wa = pltpu.async_copy(rows_a.at[slot], gs_out.at[pl.ds(base, CHUNK)],
                              sem_w[slot])
        wb = pltpu.async_copy(rows_b.at[slot], gd_out.at[pl.ds(base, CHUNK)],
                              sem_w[slot])
        pending_wb[slot] = (wa, wb)

    for p in pending_wb:
        if p is not None:
            for w in p:
                w.wait()


_decode = pl.kernel(
    _decode_body,
    out_type=(jax.ShapeDtypeStruct((L_PAD, D_FEAT), jnp.float32),
              jax.ShapeDtypeStruct((L_PAD, D_FEAT), jnp.float32)),
    mesh=_mesh,
    scratch_types=[
        pltpu.VMEM((DEC_PER_W * CHUNK,), jnp.int32),
        pltpu.VMEM((DEC_PER_W * CHUNK,), jnp.int32),
        pltpu.VMEM((2, CHUNK, D_FEAT), jnp.float32),
        pltpu.VMEM((2, CHUNK, D_FEAT), jnp.float32),
        pltpu.SemaphoreType.DMA,
        pltpu.SemaphoreType.DMA,
        pltpu.SemaphoreType.DMA,
        pltpu.SemaphoreType.DMA,
    ],
)


def _rowsum_body(gs_ref, gd_ref, out_ref):
    out_ref[...] = jnp.sum(gs_ref[...] * gd_ref[...], axis=1)


def _rowsum_tc(gs, gd):
    rows_blk = 10240
    return pl.pallas_call(
        _rowsum_body,
        grid=(L_PAD // rows_blk,),
        in_specs=[pl.BlockSpec((rows_blk, D_FEAT), lambda i: (i, 0)),
                  pl.BlockSpec((rows_blk, D_FEAT), lambda i: (i, 0))],
        out_specs=pl.BlockSpec((rows_blk,), lambda i: (i,)),
        out_shape=jax.ShapeDtypeStruct((L_PAD,), jnp.float32),
    )(gs, gd)


def _layer_tc_body(relu, agg_ref, deg_ref, w_ref, b_ref, out_ref):
    part = agg_ref[0] + agg_ref[1]
    deg = jnp.sum(deg_ref[0] + deg_ref[1], axis=1, keepdims=True)
    deg = jnp.maximum(deg, 1.0)
    aggn = part / deg
    y = jnp.dot(aggn, w_ref[...], preferred_element_type=jnp.float32) + b_ref[...]
    if relu:
        y = jnp.maximum(y, 0.0)
    out_ref[...] = y


def _layer_tc(agg_part, deg_part, w, b, relu):
    rows_blk = 2016
    grid = (N_ACC // rows_blk,)
    return pl.pallas_call(
        functools.partial(_layer_tc_body, relu),
        grid=grid,
        in_specs=[
            pl.BlockSpec((NC, rows_blk, D_FEAT), lambda i: (0, i, 0)),
            pl.BlockSpec((NC, rows_blk, D_FEAT), lambda i: (0, i, 0)),
            pl.BlockSpec((D_FEAT, D_FEAT), lambda i: (0, 0)),
            pl.BlockSpec((1, D_FEAT), lambda i: (0, 0)),
        ],
        out_specs=pl.BlockSpec((rows_blk, D_FEAT), lambda i: (i, 0)),
        out_shape=jax.ShapeDtypeStruct((N_ACC, D_FEAT), jnp.float32),
    )(agg_part, deg_part, w, b)


def kernel(x, edge_index, edge_label_index, W1, b1, W2, b2):
    epad = jnp.arange(E_PAD - N_EDGES, dtype=jnp.int32)
    lpad = jnp.arange(L_PAD - N_LABEL, dtype=jnp.int32)
    src_pad = jnp.concatenate([edge_index[0].astype(jnp.int32), epad % N_NODES])
    dst_pad = jnp.concatenate([edge_index[1].astype(jnp.int32),
                               N_NODES + epad % (N_ACC - N_NODES)])
    dst_r = dst_pad.reshape(NW * CPW, CHUNK)
    s_r = jnp.concatenate([edge_label_index[0].astype(jnp.int32), lpad % N_NODES])
    d_r = jnp.concatenate([edge_label_index[1].astype(jnp.int32), lpad % N_NODES])
    x_pad = jnp.zeros((N_ACC, D_FEAT), jnp.float32).at[:N_NODES].set(x)
    z2d = jnp.zeros((N_ACC, D_FEAT), jnp.float32)
    ones2d = jnp.zeros((CHUNK, D_FEAT), jnp.float32).at[:, 0].set(1.0)

    degp = _deg_sum(dst_r, z2d, ones2d)
    agg1p = _agg_sum(x_pad, src_pad, dst_r, z2d)
    h = _layer_tc(agg1p, degp, W1, b1.reshape(1, D_FEAT), relu=True)
    agg2p = _agg_sum(h, src_pad, dst_r, z2d)
    z = _layer_tc(agg2p, degp, W2, b2.reshape(1, D_FEAT), relu=False)
    gs, gd = _decode(z, s_r, d_r)
    scores_pad = _rowsum_tc(gs, gd)
    return scores_pad[:N_LABEL]
